# Initial kernel scaffold; baseline (speedup 1.0000x reference)
import jax
import jax.numpy as jnp
from jax.experimental import pallas as pl


def kernel(input, edge, edge_embed, a, a_2):
    raise NotImplementedError("write your pallas kernel here")


# speedup vs baseline: 3.6464x; 3.6464x over previous
import functools

import jax
import jax.numpy as jnp
from jax import lax
from jax.experimental import pallas as pl
from jax.experimental.pallas import tpu as pltpu
from jax.experimental.pallas import tpu_sc as plsc

ALPHA = 0.2


def _proj_body(x_ref, b1_ref, b2_ref, av_ref, u_ref, v_ref, s_ref):
    xb = x_ref[...]
    u = jnp.dot(xb, b1_ref[...], preferred_element_type=jnp.float32)
    v = jnp.dot(xb, b2_ref[...], preferred_element_type=jnp.float32)
    av = av_ref[...]
    su = jnp.dot(u, av, preferred_element_type=jnp.float32)
    sv = jnp.dot(v, av, preferred_element_type=jnp.float32)
    col = lax.broadcasted_iota(jnp.int32, u.shape, 1)
    sb = jnp.where(col == 0, su, 0.0) + jnp.where(col == 1, sv, 0.0)
    u_ref[...] = u
    v_ref[...] = v
    s_ref[...] = sb


def _project(x, a, a_2):
    n, in_f = x.shape
    out_f = a.shape[0]
    rows = 400
    grid = n // rows
    b1 = a[:, :in_f].T
    b2 = a[:, in_f:2 * in_f].T
    av = a_2.T
    return pl.pallas_call(
        _proj_body,
        grid=(grid,),
        in_specs=[
            pl.BlockSpec((rows, in_f), lambda i: (i, 0)),
            pl.BlockSpec((in_f, out_f), lambda i: (0, 0)),
            pl.BlockSpec((in_f, out_f), lambda i: (0, 0)),
            pl.BlockSpec((out_f, 1), lambda i: (0, 0)),
        ],
        out_specs=[
            pl.BlockSpec((rows, out_f), lambda i: (i, 0)),
            pl.BlockSpec((rows, out_f), lambda i: (i, 0)),
            pl.BlockSpec((rows, out_f), lambda i: (i, 0)),
        ],
        out_shape=[
            jax.ShapeDtypeStruct((n, out_f), jnp.float32),
            jax.ShapeDtypeStruct((n, out_f), jnp.float32),
            jax.ShapeDtypeStruct((n, out_f), jnp.float32),
        ],
    )(x, b1, b2, av)


def _sw_body(r_ref, a2_ref, a3_ref, o_ref):
    c = jnp.dot(a2_ref[...], a3_ref[...], preferred_element_type=jnp.float32)
    ri = lax.broadcasted_iota(jnp.int32, (128, 16), 0)
    ci = lax.broadcasted_iota(jnp.int32, (128, 16), 1)
    emat = (ri % 16 == ci).astype(jnp.float32)
    cmod = jnp.dot(emat, c.T, preferred_element_type=jnp.float32)
    rj = lax.broadcasted_iota(jnp.int32, (128, 8), 0)
    cj = lax.broadcasted_iota(jnp.int32, (128, 8), 1)
    m = jnp.where(rj // 16 == cj, cmod, 0.0)
    o_ref[...] = jnp.dot(r_ref[...], m, preferred_element_type=jnp.float32)


def _edge_scalar(edge_embed, a, a_2):
    e, nrela = edge_embed.shape
    in_f = (a.shape[1] - nrela) // 2
    r = edge_embed.reshape(e // 8, 8 * nrela)
    rows = 4000
    grid = (e // 8) // rows
    a3 = a[:, 2 * in_f:]
    out = pl.pallas_call(
        _sw_body,
        grid=(grid,),
        in_specs=[
            pl.BlockSpec((rows, 8 * nrela), lambda i: (i, 0)),
            pl.BlockSpec((1, a.shape[0]), lambda i: (0, 0)),
            pl.BlockSpec((a.shape[0], nrela), lambda i: (0, 0)),
        ],
        out_specs=pl.BlockSpec((rows, 8), lambda i: (i, 0)),
        out_shape=jax.ShapeDtypeStruct((e // 8, 8), jnp.float32),
    )(r, a_2, a3)
    return out.reshape(e)


def _sc_edge_kernel(n, e, out_f, nrela):
    cb = 80
    pw2 = 2 * nrela
    chunks = e // cb
    nch = chunks // 32
    rows_per_tile = -(-n // (16 * cb)) * cb
    npad = 16 * rows_per_tile

    mesh = plsc.VectorSubcoreMesh(core_axis_name="c", subcore_axis_name="s")

    @functools.partial(
        pl.kernel,
        mesh=mesh,
        out_type=[
            jax.ShapeDtypeStruct((2, npad, out_f), jnp.float32),
            jax.ShapeDtypeStruct((2, npad, pw2), jnp.float32),
        ],
        compiler_params=pltpu.CompilerParams(
            needs_layout_passes=False, use_tc_tiling_on_sc=False),
        scratch_types=[
            pltpu.VMEM((cb,), jnp.int32),
            pltpu.VMEM((cb,), jnp.int32),
            pltpu.VMEM((cb,), jnp.float32),
            pltpu.VMEM((cb,), jnp.float32),
            pltpu.VMEM((cb,), jnp.float32),
            pltpu.VMEM((cb, nrela), jnp.float32),
            pltpu.VMEM((cb, out_f), jnp.float32),
            pltpu.VMEM((cb, pw2), jnp.float32),
            pltpu.VMEM_SHARED((npad, out_f), jnp.float32),
            pltpu.VMEM_SHARED((npad, pw2), jnp.float32),
            pltpu.SemaphoreType.DMA,
        ],
    )
    def sc_kernel(v_hbm, su_hbm, sv_hbm, src_hbm, dst_hbm, sw_hbm, ee_hbm,
                  out1_hbm, out2_hbm, srcb, dstb, swb, sub, svb, eeb,
                  pay1, pay2, acc1, acc2, sem):
        cid = lax.axis_index("c")
        sid = lax.axis_index("s")

        def _zrow(r, carry):
            for k in range(out_f // 16):
                pay1[r, pl.ds(16 * k, 16)] = jnp.zeros((16,), jnp.float32)
            for k in range(pw2 // 16):
                pay2[r, pl.ds(16 * k, 16)] = jnp.zeros((16,), jnp.float32)
            return carry
        lax.fori_loop(0, cb, _zrow, 0)
        for j in range(rows_per_tile // cb):
            r0 = sid * rows_per_tile + j * cb
            pltpu.sync_copy(pay1, acc1.at[pl.ds(r0, cb)])
            pltpu.sync_copy(pay2, acc2.at[pl.ds(r0, cb)])
        plsc.subcore_barrier()

        tid = sid * 2 + cid

        def _chunk(i, carry):
            base = (tid + 32 * i) * cb
            pltpu.sync_copy(src_hbm.at[pl.ds(base, cb)], srcb)
            pltpu.sync_copy(dst_hbm.at[pl.ds(base, cb)], dstb)
            pltpu.sync_copy(sw_hbm.at[pl.ds(base, cb)], swb)
            pltpu.sync_copy(ee_hbm.at[pl.ds(base, cb)], eeb)
            cp_su = pltpu.async_copy(su_hbm.at[srcb], sub, sem)
            cp_sv = pltpu.async_copy(sv_hbm.at[dstb], svb, sem)
            cp_v = pltpu.async_copy(v_hbm.at[dstb], pay1, sem)
            cp_su.wait()
            cp_sv.wait()
            cp_v.wait()

            for g in range(cb // 16):
                s = (sub[pl.ds(16 * g, 16)] + svb[pl.ds(16 * g, 16)]
                     + swb[pl.ds(16 * g, 16)])
                w = jnp.exp(-jnp.where(s > 0, s, ALPHA * s))
                rows = lax.iota(jnp.int32, 16) + 16 * g
                cols = jnp.full((16,), nrela, jnp.int32)
                plsc.store_scatter(pay2, [rows, cols], w)
                for j in range(16):
                    ed = 16 * g + j
                    wsp = w.at[jnp.full((16,), j, jnp.int32)].get(
                        mode="promise_in_bounds")
                    for k in range(out_f // 16):
                        pay1[ed, pl.ds(16 * k, 16)] = (
                            pay1[ed, pl.ds(16 * k, 16)] * wsp)
                    pay2[ed, pl.ds(0, nrela)] = eeb[ed, :] * wsp

            pltpu.sync_copy(pay1, acc1.at[srcb], add=True)
            pltpu.sync_copy(pay2, acc2.at[srcb], add=True)
            return carry

        lax.fori_loop(0, nch, _chunk, 0)
        plsc.subcore_barrier()

        for j in range(rows_per_tile // cb):
            r0 = sid * rows_per_tile + j * cb
            pltpu.sync_copy(acc1.at[pl.ds(r0, cb)], pay1)
            pltpu.sync_copy(pay1, out1_hbm.at[cid, pl.ds(r0, cb)])
            pltpu.sync_copy(acc2.at[pl.ds(r0, cb)], pay2)
            pltpu.sync_copy(pay2, out2_hbm.at[cid, pl.ds(r0, cb)])

    return sc_kernel


def _combine_body(p1_ref, p2_ref, u_ref, a3t_ref, o_ref):
    comb1 = p1_ref[0] + p1_ref[1]
    comb2 = p2_ref[0] + p2_ref[1]
    nrela = a3t_ref.shape[0]
    acc16 = comb2[:, :nrela]
    rs = comb2[:, nrela:nrela + 1]
    num = comb1 + jnp.dot(acc16, a3t_ref[...], preferred_element_type=jnp.float32)
    safe = jnp.where(rs == 0.0, 1.0, rs)
    h = u_ref[...] + num / safe
    h = jnp.where(rs == 0.0, 0.0, h)
    o_ref[...] = jnp.where(h > 0, h, jnp.exp(h) - 1.0)


def _combine(p1, p2, u, a, nrela):
    n, out_f = u.shape
    pw2 = p2.shape[2]
    a3t = a[:, a.shape[1] - nrela:].T
    rows = 400
    grid = n // rows
    return pl.pallas_call(
        _combine_body,
        grid=(grid,),
        in_specs=[
            pl.BlockSpec((2, rows, out_f), lambda i: (0, i, 0)),
            pl.BlockSpec((2, rows, pw2), lambda i: (0, i, 0)),
            pl.BlockSpec((rows, out_f), lambda i: (i, 0)),
            pl.BlockSpec((nrela, out_f), lambda i: (0, 0)),
        ],
        out_specs=pl.BlockSpec((rows, out_f), lambda i: (i, 0)),
        out_shape=jax.ShapeDtypeStruct((n, out_f), jnp.float32),
    )(p1, p2, u, a3t)


def kernel(input, edge, edge_embed, a, a_2):
    n, in_f = input.shape
    e, nrela = edge_embed.shape
    out_f = a.shape[0]

    u, v, spack = _project(input, a, a_2)
    npad = (-(-n // (16 * 80)) * 80) * 16
    pad = ((0, npad - n),)
    su = jnp.pad(spack[:, 0], pad)
    sv = jnp.pad(spack[:, 1], pad)
    sw = _edge_scalar(edge_embed, a, a_2)

    src = edge[0]
    dst = edge[1]
    sc = _sc_edge_kernel(n, e, out_f, nrela)
    p1, p2 = sc(v, su, sv, src, dst, sw, edge_embed)
    return _combine(p1, p2, u, a, nrela)


# speedup vs baseline: 4.8610x; 1.3331x over previous
import functools

import jax
import jax.numpy as jnp
from jax import lax
from jax.experimental import pallas as pl
from jax.experimental.pallas import tpu as pltpu
from jax.experimental.pallas import tpu_sc as plsc

ALPHA = 0.2


def _proj_body(x_ref, b1_ref, b2_ref, av_ref, u_ref, v_ref, s_ref):
    xb = x_ref[...]
    u = jnp.dot(xb, b1_ref[...], preferred_element_type=jnp.float32)
    v = jnp.dot(xb, b2_ref[...], preferred_element_type=jnp.float32)
    av = av_ref[...]
    su = jnp.dot(u, av, preferred_element_type=jnp.float32)
    sv = jnp.dot(v, av, preferred_element_type=jnp.float32)
    col = lax.broadcasted_iota(jnp.int32, u.shape, 1)
    sb = jnp.where(col == 0, su, 0.0) + jnp.where(col == 1, sv, 0.0)
    u_ref[...] = u
    v_ref[...] = v
    s_ref[...] = sb


def _project(x, a, a_2):
    n, in_f = x.shape
    out_f = a.shape[0]
    rows = 400
    grid = n // rows
    b1 = a[:, :in_f].T
    b2 = a[:, in_f:2 * in_f].T
    av = a_2.T
    return pl.pallas_call(
        _proj_body,
        grid=(grid,),
        in_specs=[
            pl.BlockSpec((rows, in_f), lambda i: (i, 0)),
            pl.BlockSpec((in_f, out_f), lambda i: (0, 0)),
            pl.BlockSpec((in_f, out_f), lambda i: (0, 0)),
            pl.BlockSpec((out_f, 1), lambda i: (0, 0)),
        ],
        out_specs=[
            pl.BlockSpec((rows, out_f), lambda i: (i, 0)),
            pl.BlockSpec((rows, out_f), lambda i: (i, 0)),
            pl.BlockSpec((rows, out_f), lambda i: (i, 0)),
        ],
        out_shape=[
            jax.ShapeDtypeStruct((n, out_f), jnp.float32),
            jax.ShapeDtypeStruct((n, out_f), jnp.float32),
            jax.ShapeDtypeStruct((n, out_f), jnp.float32),
        ],
    )(x, b1, b2, av)


def _sw_body(r_ref, a2_ref, a3_ref, o_ref):
    c = jnp.dot(a2_ref[...], a3_ref[...], preferred_element_type=jnp.float32)
    ri = lax.broadcasted_iota(jnp.int32, (128, 16), 0)
    ci = lax.broadcasted_iota(jnp.int32, (128, 16), 1)
    emat = (ri % 16 == ci).astype(jnp.float32)
    cmod = jnp.dot(emat, c.T, preferred_element_type=jnp.float32)
    rj = lax.broadcasted_iota(jnp.int32, (128, 8), 0)
    cj = lax.broadcasted_iota(jnp.int32, (128, 8), 1)
    m = jnp.where(rj // 16 == cj, cmod, 0.0)
    o_ref[...] = jnp.dot(r_ref[...], m, preferred_element_type=jnp.float32)


def _edge_scalar(edge_embed, a, a_2):
    e, nrela = edge_embed.shape
    in_f = (a.shape[1] - nrela) // 2
    r = edge_embed.reshape(e // 8, 8 * nrela)
    rows = 4000
    grid = (e // 8) // rows
    a3 = a[:, 2 * in_f:]
    out = pl.pallas_call(
        _sw_body,
        grid=(grid,),
        in_specs=[
            pl.BlockSpec((rows, 8 * nrela), lambda i: (i, 0)),
            pl.BlockSpec((1, a.shape[0]), lambda i: (0, 0)),
            pl.BlockSpec((a.shape[0], nrela), lambda i: (0, 0)),
        ],
        out_specs=pl.BlockSpec((rows, 8), lambda i: (i, 0)),
        out_shape=jax.ShapeDtypeStruct((e // 8, 8), jnp.float32),
    )(r, a_2, a3)
    return out.reshape(e)


def _sc_edge_kernel(n, e, out_f, nrela):
    cb = 80
    pw2 = 2 * nrela
    chunks = e // cb
    nch = chunks // 32
    npairs = nch // 2
    tail = nch - 2 * npairs
    rows_per_tile = -(-(-(-n // 16)) // 8) * 8
    npad = 16 * rows_per_tile

    mesh = plsc.VectorSubcoreMesh(core_axis_name="c", subcore_axis_name="s")

    @functools.partial(
        pl.kernel,
        mesh=mesh,
        out_type=[
            jax.ShapeDtypeStruct((2, npad, out_f), jnp.float32),
            jax.ShapeDtypeStruct((2, npad, pw2), jnp.float32),
        ],
        compiler_params=pltpu.CompilerParams(
            needs_layout_passes=False, use_tc_tiling_on_sc=False),
        scratch_types=[
            pltpu.VMEM((2, cb), jnp.int32),
            pltpu.VMEM((2, cb), jnp.int32),
            pltpu.VMEM((2, cb), jnp.float32),
            pltpu.VMEM((2, cb), jnp.float32),
            pltpu.VMEM((2, cb), jnp.float32),
            pltpu.VMEM((2, cb, nrela), jnp.float32),
            pltpu.VMEM((2, cb, out_f), jnp.float32),
            pltpu.VMEM((2, cb, pw2), jnp.float32),
            pltpu.SemaphoreType.DMA,
            pltpu.SemaphoreType.DMA,
            pltpu.SemaphoreType.DMA,
            pltpu.SemaphoreType.DMA,
            pltpu.SemaphoreType.DMA,
            pltpu.SemaphoreType.DMA,
            pltpu.VMEM_SHARED((npad, out_f), jnp.float32),
            pltpu.VMEM_SHARED((npad, pw2), jnp.float32),
        ],
    )
    def sc_kernel(v_hbm, su_hbm, sv_hbm, src_hbm, dst_hbm, sw_hbm, ee_hbm,
                  out1_hbm, out2_hbm, srcb, dstb, swb, sub, svb, eeb,
                  pay1, pay2, semA0, semA1, semB0, semB1, semS0, semS1,
                  acc1, acc2):
        cid = lax.axis_index("c")
        sid = lax.axis_index("s")
        semA = (semA0, semA1)
        semB = (semB0, semB1)
        semS = (semS0, semS1)

        def _zrow(r, carry):
            for k in range(out_f // 16):
                pay1[0, r, pl.ds(16 * k, 16)] = jnp.zeros((16,), jnp.float32)
            for k in range(pw2 // 16):
                pay2[0, r, pl.ds(16 * k, 16)] = jnp.zeros((16,), jnp.float32)
            return carry
        lax.fori_loop(0, cb, _zrow, 0)
        nzfull = rows_per_tile // cb
        zrem = rows_per_tile - nzfull * cb
        for j in range(nzfull):
            r0 = sid * rows_per_tile + j * cb
            pltpu.sync_copy(pay1.at[0], acc1.at[pl.ds(r0, cb)])
            pltpu.sync_copy(pay2.at[0], acc2.at[pl.ds(r0, cb)])
        if zrem:
            r0 = sid * rows_per_tile + nzfull * cb
            pltpu.sync_copy(pay1.at[0, pl.ds(0, zrem)], acc1.at[pl.ds(r0, zrem)])
            pltpu.sync_copy(pay2.at[0, pl.ds(0, zrem)], acc2.at[pl.ds(r0, zrem)])
        plsc.subcore_barrier()

        tid = sid * 2 + cid

        def _issue_a(k, b):
            base = (tid + 32 * k) * cb
            pltpu.async_copy(src_hbm.at[pl.ds(base, cb)], srcb.at[b], semA[b])
            pltpu.async_copy(dst_hbm.at[pl.ds(base, cb)], dstb.at[b], semA[b])
            pltpu.async_copy(sw_hbm.at[pl.ds(base, cb)], swb.at[b], semA[b])
            pltpu.async_copy(ee_hbm.at[pl.ds(base, cb)], eeb.at[b], semA[b])

        def _wait_a(b):
            pltpu.make_async_copy(src_hbm.at[pl.ds(0, cb)], srcb.at[b], semA[b]).wait()
            pltpu.make_async_copy(dst_hbm.at[pl.ds(0, cb)], dstb.at[b], semA[b]).wait()
            pltpu.make_async_copy(sw_hbm.at[pl.ds(0, cb)], swb.at[b], semA[b]).wait()
            pltpu.make_async_copy(ee_hbm.at[pl.ds(0, cb)], eeb.at[b], semA[b]).wait()

        def _issue_b(b):
            pltpu.async_copy(su_hbm.at[srcb.at[b]], sub.at[b], semB[b])
            pltpu.async_copy(sv_hbm.at[dstb.at[b]], svb.at[b], semB[b])
            pltpu.async_copy(v_hbm.at[dstb.at[b]], pay1.at[b], semB[b])

        def _wait_b(b):
            pltpu.make_async_copy(su_hbm.at[srcb.at[b]], sub.at[b], semB[b]).wait()
            pltpu.make_async_copy(sv_hbm.at[dstb.at[b]], svb.at[b], semB[b]).wait()
            pltpu.make_async_copy(v_hbm.at[dstb.at[b]], pay1.at[b], semB[b]).wait()

        def _issue_s(b):
            pltpu.async_copy(pay1.at[b], acc1.at[srcb.at[b]], semS[b], add=True)
            pltpu.async_copy(pay2.at[b], acc2.at[srcb.at[b]], semS[b], add=True)

        def _wait_s(b):
            pltpu.make_async_copy(pay1.at[b], acc1.at[srcb.at[b]], semS[b]).wait()
            pltpu.make_async_copy(pay2.at[b], acc2.at[srcb.at[b]], semS[b]).wait()

        def _compute(b):
            for g in range(cb // 16):
                s = (sub[b, pl.ds(16 * g, 16)] + svb[b, pl.ds(16 * g, 16)]
                     + swb[b, pl.ds(16 * g, 16)])
                w = jnp.exp(-jnp.where(s > 0, s, ALPHA * s))
                rows = lax.iota(jnp.int32, 16) + 16 * g
                cols = jnp.full((16,), nrela, jnp.int32)
                plsc.store_scatter(pay2.at[b], [rows, cols], w)
                for j in range(16):
                    ed = 16 * g + j
                    wsp = w.at[jnp.full((16,), j, jnp.int32)].get(
                        mode="promise_in_bounds")
                    for k in range(out_f // 16):
                        pay1[b, ed, pl.ds(16 * k, 16)] = (
                            pay1[b, ed, pl.ds(16 * k, 16)] * wsp)
                    pay2[b, ed, pl.ds(0, nrela)] = eeb[b, ed, :] * wsp

        def _pair(i, carry):
            @pl.when(i > 0)
            def _():
                _wait_s(0)
                _wait_s(1)
            _issue_a(2 * i, 0)
            _issue_a(2 * i + 1, 1)
            _wait_a(0)
            _issue_b(0)
            _wait_a(1)
            _issue_b(1)
            _wait_b(0)
            _compute(0)
            _issue_s(0)
            _wait_b(1)
            _compute(1)
            _issue_s(1)
            return carry

        lax.fori_loop(0, npairs, _pair, 0)
        _wait_s(0)
        if tail:
            _issue_a(2 * npairs, 0)
            _wait_a(0)
            _issue_b(0)
            _wait_b(0)
            _compute(0)
            _issue_s(0)
        _wait_s(1)
        if tail:
            _wait_s(0)
        plsc.subcore_barrier()

        for j in range(nzfull):
            r0 = sid * rows_per_tile + j * cb
            pltpu.sync_copy(acc1.at[pl.ds(r0, cb)], pay1.at[0])
            pltpu.sync_copy(pay1.at[0], out1_hbm.at[cid, pl.ds(r0, cb)])
            pltpu.sync_copy(acc2.at[pl.ds(r0, cb)], pay2.at[0])
            pltpu.sync_copy(pay2.at[0], out2_hbm.at[cid, pl.ds(r0, cb)])
        if zrem:
            r0 = sid * rows_per_tile + nzfull * cb
            pltpu.sync_copy(acc1.at[pl.ds(r0, zrem)], pay1.at[0, pl.ds(0, zrem)])
            pltpu.sync_copy(pay1.at[0, pl.ds(0, zrem)],
                            out1_hbm.at[cid, pl.ds(r0, zrem)])
            pltpu.sync_copy(acc2.at[pl.ds(r0, zrem)], pay2.at[0, pl.ds(0, zrem)])
            pltpu.sync_copy(pay2.at[0, pl.ds(0, zrem)],
                            out2_hbm.at[cid, pl.ds(r0, zrem)])

    return sc_kernel


def _combine_body(p1_ref, p2_ref, u_ref, a3t_ref, o_ref):
    comb1 = p1_ref[0] + p1_ref[1]
    comb2 = p2_ref[0] + p2_ref[1]
    nrela = a3t_ref.shape[0]
    acc16 = comb2[:, :nrela]
    rs = comb2[:, nrela:nrela + 1]
    num = comb1 + jnp.dot(acc16, a3t_ref[...], preferred_element_type=jnp.float32)
    safe = jnp.where(rs == 0.0, 1.0, rs)
    h = u_ref[...] + num / safe
    h = jnp.where(rs == 0.0, 0.0, h)
    o_ref[...] = jnp.where(h > 0, h, jnp.exp(h) - 1.0)


def _combine(p1, p2, u, a, nrela):
    n, out_f = u.shape
    pw2 = p2.shape[2]
    a3t = a[:, a.shape[1] - nrela:].T
    rows = 400
    grid = n // rows
    return pl.pallas_call(
        _combine_body,
        grid=(grid,),
        in_specs=[
            pl.BlockSpec((2, rows, out_f), lambda i: (0, i, 0)),
            pl.BlockSpec((2, rows, pw2), lambda i: (0, i, 0)),
            pl.BlockSpec((rows, out_f), lambda i: (i, 0)),
            pl.BlockSpec((nrela, out_f), lambda i: (0, 0)),
        ],
        out_specs=pl.BlockSpec((rows, out_f), lambda i: (i, 0)),
        out_shape=jax.ShapeDtypeStruct((n, out_f), jnp.float32),
    )(p1, p2, u, a3t)


def kernel(input, edge, edge_embed, a, a_2):
    n, in_f = input.shape
    e, nrela = edge_embed.shape
    out_f = a.shape[0]

    u, v, spack = _project(input, a, a_2)
    npad = 16 * (-(-(-(-n // 16)) // 8) * 8)
    pad = ((0, npad - n),)
    su = jnp.pad(spack[:, 0], pad)
    sv = jnp.pad(spack[:, 1], pad)
    sw = _edge_scalar(edge_embed, a, a_2)

    src = edge[0]
    dst = edge[1]
    sc = _sc_edge_kernel(n, e, out_f, nrela)
    p1, p2 = sc(v, su, sv, src, dst, sw, edge_embed)
    return _combine(p1, p2, u, a, nrela)
